# Initial kernel scaffold; baseline (speedup 1.0000x reference)
import jax
import jax.numpy as jnp
from jax.experimental import pallas as pl


def kernel(x, edge_index, W1, b1, W2, b2):
    raise NotImplementedError("write your pallas kernel here")


# speedup vs baseline: 12.0599x; 12.0599x over previous
import functools

import jax
import jax.numpy as jnp
from jax import lax
from jax.experimental import pallas as pl
from jax.experimental.pallas import tpu as pltpu
from jax.experimental.pallas import tpu_sc as plsc

N = 10000
E = 320000
D_FEAT = 128
EMBED = 128
N_CLASSES = 18
CPAD = 32

NCORES = 2
NSUB = 16
NW = NCORES * NSUB
CHUNK = 128
CPT = 80
EPT = CHUNK * CPT
EPAD = NW * EPT
EROWS = EPAD // CHUNK
NPAD = 10240
RPT = NPAD // NSUB

RB = 400
NBLK = N // RB

_mesh = plsc.VectorSubcoreMesh(core_axis_name="c", subcore_axis_name="s")


def _make_seg(width):
  @functools.partial(
      pl.kernel,
      mesh=_mesh,
      compiler_params=pltpu.CompilerParams(use_tc_tiling_on_sc=False),
      out_type=jax.ShapeDtypeStruct((NCORES, NPAD, width), jnp.float32),
      scratch_types=[
          pltpu.VMEM((CPT, CHUNK), jnp.int32),
          pltpu.VMEM((CPT, CHUNK), jnp.int32),
          pltpu.VMEM((CHUNK, width), jnp.float32),
          pltpu.VMEM_SHARED((NPAD, width), jnp.float32),
      ],
  )
  def seg(y_hbm, src_hbm, dst_hbm, zero_hbm, out_hbm, idx_s, idx_d, data, acc):
    c = lax.axis_index("c")
    s = lax.axis_index("s")
    w = c * NSUB + s
    pltpu.sync_copy(zero_hbm, acc.at[pl.ds(s * RPT, RPT)])
    pltpu.sync_copy(src_hbm.at[pl.ds(w * CPT, CPT)], idx_s)
    pltpu.sync_copy(dst_hbm.at[pl.ds(w * CPT, CPT)], idx_d)
    plsc.subcore_barrier()

    @pl.loop(0, CPT)
    def _(j):
      pltpu.sync_copy(y_hbm.at[idx_s.at[j]], data)
      pltpu.sync_copy(data, acc.at[idx_d.at[j]], add=True)

    plsc.subcore_barrier()
    pltpu.sync_copy(acc.at[pl.ds(s * RPT, RPT)],
                    out_hbm.at[c, pl.ds(s * RPT, RPT)])

  return seg


_seg128 = _make_seg(EMBED)
_seg32 = _make_seg(CPAD)


@functools.partial(
    pl.kernel,
    mesh=_mesh,
    compiler_params=pltpu.CompilerParams(use_tc_tiling_on_sc=False),
    out_type=jax.ShapeDtypeStruct((NCORES, NPAD, 16), jnp.float32),
    scratch_types=[
        pltpu.VMEM((CPT, CHUNK), jnp.int32),
        pltpu.VMEM((CHUNK, 16), jnp.float32),
        pltpu.VMEM_SHARED((NPAD, 16), jnp.float32),
    ],
)
def _deg(ones_hbm, dst_hbm, zero_hbm, out_hbm, idx_d, ones_v, acc):
  c = lax.axis_index("c")
  s = lax.axis_index("s")
  w = c * NSUB + s
  pltpu.sync_copy(zero_hbm, acc.at[pl.ds(s * RPT, RPT)])
  pltpu.sync_copy(ones_hbm, ones_v)
  pltpu.sync_copy(dst_hbm.at[pl.ds(w * CPT, CPT)], idx_d)
  plsc.subcore_barrier()

  @pl.loop(0, CPT)
  def _(j):
    pltpu.sync_copy(ones_v, acc.at[idx_d.at[j]], add=True)

  plsc.subcore_barrier()
  pltpu.sync_copy(acc.at[pl.ds(s * RPT, RPT)],
                  out_hbm.at[c, pl.ds(s * RPT, RPT)])


def _tc_a(x, W1, degp):
  def body(x_ref, w_ref, d_ref, dis_ref, y_ref):
    deg = d_ref[0] + d_ref[1] + 1.0
    dis = lax.rsqrt(deg)
    dis_ref[...] = dis
    xw = jnp.dot(x_ref[...], w_ref[...], preferred_element_type=jnp.float32)
    y_ref[...] = xw * dis[:, 0:1]

  return pl.pallas_call(
      body,
      grid=(NBLK,),
      in_specs=[
          pl.BlockSpec((RB, D_FEAT), lambda i: (i, 0)),
          pl.BlockSpec((D_FEAT, EMBED), lambda i: (0, 0)),
          pl.BlockSpec((NCORES, RB, 16), lambda i: (0, i, 0)),
      ],
      out_specs=[
          pl.BlockSpec((RB, 16), lambda i: (i, 0)),
          pl.BlockSpec((RB, EMBED), lambda i: (i, 0)),
      ],
      out_shape=[
          jax.ShapeDtypeStruct((N, 16), jnp.float32),
          jax.ShapeDtypeStruct((N, EMBED), jnp.float32),
      ],
  )(x, W1, degp)


def _tc_b(acc1, Y1, dis, b1, W2p):
  def body(a_ref, y1_ref, dis_ref, b1_ref, w2_ref, y2_ref):
    dis0 = dis_ref[:, 0:1]
    h = (a_ref[0] + a_ref[1] + y1_ref[...]) * dis0 + b1_ref[...]
    h = jnp.maximum(h, 0.0)
    y2_ref[...] = jnp.dot(h, w2_ref[...],
                          preferred_element_type=jnp.float32) * dis0

  return pl.pallas_call(
      body,
      grid=(NBLK,),
      in_specs=[
          pl.BlockSpec((NCORES, RB, EMBED), lambda i: (0, i, 0)),
          pl.BlockSpec((RB, EMBED), lambda i: (i, 0)),
          pl.BlockSpec((RB, 16), lambda i: (i, 0)),
          pl.BlockSpec((1, EMBED), lambda i: (0, 0)),
          pl.BlockSpec((EMBED, CPAD), lambda i: (0, 0)),
      ],
      out_specs=pl.BlockSpec((RB, CPAD), lambda i: (i, 0)),
      out_shape=jax.ShapeDtypeStruct((N, CPAD), jnp.float32),
  )(acc1, Y1, dis, b1, W2p)


def _tc_c(acc2, Y2, dis, b2p):
  def body(a_ref, y2_ref, dis_ref, b2_ref, lp_ref, o_ref):
    dis0 = dis_ref[:, 0:1]
    o = (a_ref[0] + a_ref[1] + y2_ref[...]) * dis0 + b2_ref[...]
    logits = o[:, :N_CLASSES]
    m = jnp.max(logits, axis=1, keepdims=True)
    lse = jnp.log(jnp.sum(jnp.exp(logits - m), axis=1, keepdims=True)) + m
    o_ref[...] = logits
    lp_ref[...] = logits - lse

  return pl.pallas_call(
      body,
      grid=(NBLK,),
      in_specs=[
          pl.BlockSpec((NCORES, RB, CPAD), lambda i: (0, i, 0)),
          pl.BlockSpec((RB, CPAD), lambda i: (i, 0)),
          pl.BlockSpec((RB, 16), lambda i: (i, 0)),
          pl.BlockSpec((1, CPAD), lambda i: (0, 0)),
      ],
      out_specs=[
          pl.BlockSpec((RB, N_CLASSES), lambda i: (i, 0)),
          pl.BlockSpec((RB, N_CLASSES), lambda i: (i, 0)),
      ],
      out_shape=[
          jax.ShapeDtypeStruct((N, N_CLASSES), jnp.float32),
          jax.ShapeDtypeStruct((N, N_CLASSES), jnp.float32),
      ],
  )(acc2, Y2, dis, b2p)


def kernel(x, edge_index, W1, b1, W2, b2):
  src = edge_index[0]
  dst = edge_index[1]
  pad = EPAD - E
  srcp = jnp.concatenate(
      [src, jnp.zeros((pad,), jnp.int32)]).reshape(EROWS, CHUNK)
  dstp = jnp.concatenate(
      [dst, jnp.full((pad,), N, jnp.int32)]).reshape(EROWS, CHUNK)

  ones16 = jnp.ones((CHUNK, 16), jnp.float32)
  z16 = jnp.zeros((RPT, 16), jnp.float32)
  z128 = jnp.zeros((RPT, EMBED), jnp.float32)
  z32 = jnp.zeros((RPT, CPAD), jnp.float32)

  degp = _deg(ones16, dstp, z16)
  dis, Y1 = _tc_a(x, W1, degp[:, :N, :])
  acc1 = _seg128(Y1, srcp, dstp, z128)[:, :N, :]

  W2p = jnp.pad(W2, ((0, 0), (0, CPAD - N_CLASSES)))
  b2p = jnp.pad(b2, (0, CPAD - N_CLASSES)).reshape(1, CPAD)
  Y2 = _tc_b(acc1, Y1, dis, b1.reshape(1, EMBED), W2p)
  acc2 = _seg32(Y2, srcp, dstp, z32)[:, :N, :]

  logp, out = _tc_c(acc2, Y2, dis, b2p)
  return (logp, out)


# speedup vs baseline: 13.5581x; 1.1242x over previous
import functools

import jax
import jax.numpy as jnp
from jax import lax
from jax.experimental import pallas as pl
from jax.experimental.pallas import tpu as pltpu
from jax.experimental.pallas import tpu_sc as plsc

N = 10000
E = 320000
D_FEAT = 128
EMBED = 128
N_CLASSES = 18
CPAD = 32

NCORES = 2
NSUB = 16
NW = NCORES * NSUB
CHUNK = 64
CPT = 160
EPT = CHUNK * CPT
EPAD = NW * EPT
EROWS = EPAD // CHUNK
NPAD = 10240
RPT = NPAD // NSUB

RB = 400
NBLK = N // RB

_mesh = plsc.VectorSubcoreMesh(core_axis_name="c", subcore_axis_name="s")


def _make_seg(width):
  @functools.partial(
      pl.kernel,
      mesh=_mesh,
      compiler_params=pltpu.CompilerParams(use_tc_tiling_on_sc=False),
      out_type=jax.ShapeDtypeStruct((NCORES, NPAD, width), jnp.float32),
      scratch_types=[
          pltpu.VMEM((CPT, CHUNK), jnp.int32),
          pltpu.VMEM((CPT, CHUNK), jnp.int32),
          pltpu.VMEM((CHUNK, width), jnp.float32),
          pltpu.VMEM((CHUNK, width), jnp.float32),
          pltpu.SemaphoreType.DMA,
          pltpu.SemaphoreType.DMA,
          pltpu.VMEM_SHARED((NPAD, width), jnp.float32),
      ],
  )
  def seg(y_hbm, src_hbm, dst_hbm, zero_hbm, out_hbm, idx_s, idx_d,
          data_a, data_b, sem_a, sem_b, acc):
    c = lax.axis_index("c")
    s = lax.axis_index("s")
    w = c * NSUB + s
    pltpu.sync_copy(zero_hbm, acc.at[pl.ds(s * RPT, RPT)])
    pltpu.sync_copy(src_hbm.at[pl.ds(w * CPT, CPT)], idx_s)
    pltpu.sync_copy(dst_hbm.at[pl.ds(w * CPT, CPT)], idx_d)
    plsc.subcore_barrier()

    def gather(j, buf, sem):
      return pltpu.make_async_copy(y_hbm.at[idx_s.at[j]], buf, sem)

    gather(0, data_a, sem_a).start()
    gather(1, data_b, sem_b).start()

    @pl.loop(0, CPT, step=2)
    def _(j):
      gather(j, data_a, sem_a).wait()
      pltpu.sync_copy(data_a, acc.at[idx_d.at[j]], add=True)

      @pl.when(j + 2 < CPT)
      def _():
        gather(j + 2, data_a, sem_a).start()

      gather(j + 1, data_b, sem_b).wait()
      pltpu.sync_copy(data_b, acc.at[idx_d.at[j + 1]], add=True)

      @pl.when(j + 3 < CPT)
      def _():
        gather(j + 3, data_b, sem_b).start()

    plsc.subcore_barrier()
    pltpu.sync_copy(acc.at[pl.ds(s * RPT, RPT)],
                    out_hbm.at[c, pl.ds(s * RPT, RPT)])

  return seg


_seg128 = _make_seg(EMBED)
_seg32 = _make_seg(CPAD)


@functools.partial(
    pl.kernel,
    mesh=_mesh,
    compiler_params=pltpu.CompilerParams(use_tc_tiling_on_sc=False),
    out_type=jax.ShapeDtypeStruct((NCORES, NPAD, 16), jnp.float32),
    scratch_types=[
        pltpu.VMEM((CPT, CHUNK), jnp.int32),
        pltpu.VMEM((CHUNK, 16), jnp.float32),
        pltpu.VMEM_SHARED((NPAD, 16), jnp.float32),
    ],
)
def _deg(ones_hbm, dst_hbm, zero_hbm, out_hbm, idx_d, ones_v, acc):
  c = lax.axis_index("c")
  s = lax.axis_index("s")
  w = c * NSUB + s
  pltpu.sync_copy(zero_hbm, acc.at[pl.ds(s * RPT, RPT)])
  pltpu.sync_copy(ones_hbm, ones_v)
  pltpu.sync_copy(dst_hbm.at[pl.ds(w * CPT, CPT)], idx_d)
  plsc.subcore_barrier()

  @pl.loop(0, CPT)
  def _(j):
    pltpu.sync_copy(ones_v, acc.at[idx_d.at[j]], add=True)

  plsc.subcore_barrier()
  pltpu.sync_copy(acc.at[pl.ds(s * RPT, RPT)],
                  out_hbm.at[c, pl.ds(s * RPT, RPT)])


def _tc_a(x, W1, degp):
  def body(x_ref, w_ref, d_ref, dis_ref, y_ref):
    deg = d_ref[0] + d_ref[1] + 1.0
    dis = lax.rsqrt(deg)
    dis_ref[...] = dis
    xw = jnp.dot(x_ref[...], w_ref[...], preferred_element_type=jnp.float32)
    y_ref[...] = xw * dis[:, 0:1]

  return pl.pallas_call(
      body,
      grid=(NBLK,),
      in_specs=[
          pl.BlockSpec((RB, D_FEAT), lambda i: (i, 0)),
          pl.BlockSpec((D_FEAT, EMBED), lambda i: (0, 0)),
          pl.BlockSpec((NCORES, RB, 16), lambda i: (0, i, 0)),
      ],
      out_specs=[
          pl.BlockSpec((RB, 16), lambda i: (i, 0)),
          pl.BlockSpec((RB, EMBED), lambda i: (i, 0)),
      ],
      out_shape=[
          jax.ShapeDtypeStruct((N, 16), jnp.float32),
          jax.ShapeDtypeStruct((N, EMBED), jnp.float32),
      ],
  )(x, W1, degp)


def _tc_b(acc1, Y1, dis, b1, W2p):
  def body(a_ref, y1_ref, dis_ref, b1_ref, w2_ref, y2_ref):
    dis0 = dis_ref[:, 0:1]
    h = (a_ref[0] + a_ref[1] + y1_ref[...]) * dis0 + b1_ref[...]
    h = jnp.maximum(h, 0.0)
    y2_ref[...] = jnp.dot(h, w2_ref[...],
                          preferred_element_type=jnp.float32) * dis0

  return pl.pallas_call(
      body,
      grid=(NBLK,),
      in_specs=[
          pl.BlockSpec((NCORES, RB, EMBED), lambda i: (0, i, 0)),
          pl.BlockSpec((RB, EMBED), lambda i: (i, 0)),
          pl.BlockSpec((RB, 16), lambda i: (i, 0)),
          pl.BlockSpec((1, EMBED), lambda i: (0, 0)),
          pl.BlockSpec((EMBED, CPAD), lambda i: (0, 0)),
      ],
      out_specs=pl.BlockSpec((RB, CPAD), lambda i: (i, 0)),
      out_shape=jax.ShapeDtypeStruct((N, CPAD), jnp.float32),
  )(acc1, Y1, dis, b1, W2p)


def _tc_c(acc2, Y2, dis, b2p):
  def body(a_ref, y2_ref, dis_ref, b2_ref, lp_ref, o_ref):
    dis0 = dis_ref[:, 0:1]
    o = (a_ref[0] + a_ref[1] + y2_ref[...]) * dis0 + b2_ref[...]
    logits = o[:, :N_CLASSES]
    m = jnp.max(logits, axis=1, keepdims=True)
    lse = jnp.log(jnp.sum(jnp.exp(logits - m), axis=1, keepdims=True)) + m
    o_ref[...] = logits
    lp_ref[...] = logits - lse

  return pl.pallas_call(
      body,
      grid=(NBLK,),
      in_specs=[
          pl.BlockSpec((NCORES, RB, CPAD), lambda i: (0, i, 0)),
          pl.BlockSpec((RB, CPAD), lambda i: (i, 0)),
          pl.BlockSpec((RB, 16), lambda i: (i, 0)),
          pl.BlockSpec((1, CPAD), lambda i: (0, 0)),
      ],
      out_specs=[
          pl.BlockSpec((RB, N_CLASSES), lambda i: (i, 0)),
          pl.BlockSpec((RB, N_CLASSES), lambda i: (i, 0)),
      ],
      out_shape=[
          jax.ShapeDtypeStruct((N, N_CLASSES), jnp.float32),
          jax.ShapeDtypeStruct((N, N_CLASSES), jnp.float32),
      ],
  )(acc2, Y2, dis, b2p)


def kernel(x, edge_index, W1, b1, W2, b2):
  src = edge_index[0]
  dst = edge_index[1]
  pad = EPAD - E
  srcp = jnp.concatenate(
      [src, jnp.zeros((pad,), jnp.int32)]).reshape(EROWS, CHUNK)
  dstp = jnp.concatenate(
      [dst, jnp.full((pad,), N, jnp.int32)]).reshape(EROWS, CHUNK)

  ones16 = jnp.ones((CHUNK, 16), jnp.float32)
  z16 = jnp.zeros((RPT, 16), jnp.float32)
  z128 = jnp.zeros((RPT, EMBED), jnp.float32)
  z32 = jnp.zeros((RPT, CPAD), jnp.float32)

  degp = _deg(ones16, dstp, z16)
  dis, Y1 = _tc_a(x, W1, degp[:, :N, :])
  acc1 = _seg128(Y1, srcp, dstp, z128)[:, :N, :]

  W2p = jnp.pad(W2, ((0, 0), (0, CPAD - N_CLASSES)))
  b2p = jnp.pad(b2, (0, CPAD - N_CLASSES)).reshape(1, CPAD)
  Y2 = _tc_b(acc1, Y1, dis, b1.reshape(1, EMBED), W2p)
  acc2 = _seg32(Y2, srcp, dstp, z32)[:, :N, :]

  logp, out = _tc_c(acc2, Y2, dis, b2p)
  return (logp, out)


# speedup vs baseline: 15.9423x; 1.1759x over previous
import functools

import jax
import jax.numpy as jnp
from jax import lax
from jax.experimental import pallas as pl
from jax.experimental.pallas import tpu as pltpu
from jax.experimental.pallas import tpu_sc as plsc

N = 10000
E = 320000
D_FEAT = 128
EMBED = 128
N_CLASSES = 18
CPAD = 32

NCORES = 2
NSUB = 16
NW = NCORES * NSUB
CHUNK = 64
CPT = 159
EPT = CHUNK * CPT
EPAD = NW * EPT
EROWS = EPAD // CHUNK
NPAD = 10240
RPT = NPAD // NSUB

RB = 400
NBLK = N // RB

_mesh = plsc.VectorSubcoreMesh(core_axis_name="c", subcore_axis_name="s")


def _make_seg(width):
  @functools.partial(
      pl.kernel,
      mesh=_mesh,
      compiler_params=pltpu.CompilerParams(use_tc_tiling_on_sc=False),
      out_type=jax.ShapeDtypeStruct((NCORES, NPAD, width), jnp.float32),
      scratch_types=[
          pltpu.VMEM((CPT, CHUNK), jnp.int32),
          pltpu.VMEM((CPT, CHUNK), jnp.int32),
          pltpu.VMEM((CHUNK, width), jnp.float32),
          pltpu.VMEM((CHUNK, width), jnp.float32),
          pltpu.VMEM((CHUNK, width), jnp.float32),
          pltpu.SemaphoreType.DMA,
          pltpu.SemaphoreType.DMA,
          pltpu.SemaphoreType.DMA,
          pltpu.SemaphoreType.DMA,
          pltpu.SemaphoreType.DMA,
          pltpu.SemaphoreType.DMA,
          pltpu.VMEM_SHARED((NPAD, width), jnp.float32),
      ],
  )
  def seg(y_hbm, src_hbm, dst_hbm, zero_hbm, out_hbm, idx_s, idx_d,
          d0, d1, d2, g0, g1, g2, s0, s1, s2, acc):
    c = lax.axis_index("c")
    s = lax.axis_index("s")
    w = c * NSUB + s
    pltpu.sync_copy(zero_hbm, acc.at[pl.ds(s * RPT, RPT)])
    pltpu.sync_copy(src_hbm.at[pl.ds(w * CPT, CPT)], idx_s)
    pltpu.sync_copy(dst_hbm.at[pl.ds(w * CPT, CPT)], idx_d)
    plsc.subcore_barrier()

    bufs = (d0, d1, d2)
    gsem = (g0, g1, g2)
    ssem = (s0, s1, s2)

    def gather(j, p):
      return pltpu.make_async_copy(y_hbm.at[idx_s.at[j]], bufs[p], gsem[p])

    def scatter_start(j, p):
      pltpu.async_copy(bufs[p], acc.at[idx_d.at[j]], ssem[p], add=True)

    def scatter_wait(j, p):
      pltpu.make_async_copy(bufs[p], acc.at[idx_d.at[j]], ssem[p]).wait()

    gather(0, 0).start()
    gather(1, 1).start()

    @pl.loop(0, CPT, step=3)
    def _(j):
      for p in range(3):
        jj = j + p
        gather(jj, p).wait()
        scatter_start(jj, p)
        q = (p + 2) % 3

        @pl.when(jj >= 1)
        def _():
          scatter_wait(jj - 1, q)

        @pl.when(jj + 2 < CPT)
        def _():
          gather(jj + 2, q).start()

    scatter_wait(CPT - 1, (CPT - 1) % 3)
    plsc.subcore_barrier()
    pltpu.sync_copy(acc.at[pl.ds(s * RPT, RPT)],
                    out_hbm.at[c, pl.ds(s * RPT, RPT)])

  return seg


_seg128 = _make_seg(EMBED)
_seg32 = _make_seg(CPAD)


@functools.partial(
    pl.kernel,
    mesh=_mesh,
    compiler_params=pltpu.CompilerParams(use_tc_tiling_on_sc=False),
    out_type=jax.ShapeDtypeStruct((NCORES, NPAD, 16), jnp.float32),
    scratch_types=[
        pltpu.VMEM((CPT, CHUNK), jnp.int32),
        pltpu.VMEM((CHUNK, 16), jnp.float32),
        pltpu.VMEM_SHARED((NPAD, 16), jnp.float32),
    ],
)
def _deg(ones_hbm, dst_hbm, zero_hbm, out_hbm, idx_d, ones_v, acc):
  c = lax.axis_index("c")
  s = lax.axis_index("s")
  w = c * NSUB + s
  pltpu.sync_copy(zero_hbm, acc.at[pl.ds(s * RPT, RPT)])
  pltpu.sync_copy(ones_hbm, ones_v)
  pltpu.sync_copy(dst_hbm.at[pl.ds(w * CPT, CPT)], idx_d)
  plsc.subcore_barrier()

  @pl.loop(0, CPT)
  def _(j):
    pltpu.sync_copy(ones_v, acc.at[idx_d.at[j]], add=True)

  plsc.subcore_barrier()
  pltpu.sync_copy(acc.at[pl.ds(s * RPT, RPT)],
                  out_hbm.at[c, pl.ds(s * RPT, RPT)])


def _tc_a(x, W1, degp):
  def body(x_ref, w_ref, d_ref, dis_ref, y_ref):
    deg = d_ref[0] + d_ref[1] + 1.0
    dis = lax.rsqrt(deg)
    dis_ref[...] = dis
    xw = jnp.dot(x_ref[...], w_ref[...], preferred_element_type=jnp.float32)
    y_ref[...] = xw * dis[:, 0:1]

  return pl.pallas_call(
      body,
      grid=(NBLK,),
      in_specs=[
          pl.BlockSpec((RB, D_FEAT), lambda i: (i, 0)),
          pl.BlockSpec((D_FEAT, EMBED), lambda i: (0, 0)),
          pl.BlockSpec((NCORES, RB, 16), lambda i: (0, i, 0)),
      ],
      out_specs=[
          pl.BlockSpec((RB, 16), lambda i: (i, 0)),
          pl.BlockSpec((RB, EMBED), lambda i: (i, 0)),
      ],
      out_shape=[
          jax.ShapeDtypeStruct((N, 16), jnp.float32),
          jax.ShapeDtypeStruct((N, EMBED), jnp.float32),
      ],
  )(x, W1, degp)


def _tc_b(acc1, Y1, dis, b1, W2p):
  def body(a_ref, y1_ref, dis_ref, b1_ref, w2_ref, y2_ref):
    dis0 = dis_ref[:, 0:1]
    h = (a_ref[0] + a_ref[1] + y1_ref[...]) * dis0 + b1_ref[...]
    h = jnp.maximum(h, 0.0)
    y2_ref[...] = jnp.dot(h, w2_ref[...],
                          preferred_element_type=jnp.float32) * dis0

  return pl.pallas_call(
      body,
      grid=(NBLK,),
      in_specs=[
          pl.BlockSpec((NCORES, RB, EMBED), lambda i: (0, i, 0)),
          pl.BlockSpec((RB, EMBED), lambda i: (i, 0)),
          pl.BlockSpec((RB, 16), lambda i: (i, 0)),
          pl.BlockSpec((1, EMBED), lambda i: (0, 0)),
          pl.BlockSpec((EMBED, CPAD), lambda i: (0, 0)),
      ],
      out_specs=pl.BlockSpec((RB, CPAD), lambda i: (i, 0)),
      out_shape=jax.ShapeDtypeStruct((N, CPAD), jnp.float32),
  )(acc1, Y1, dis, b1, W2p)


def _tc_c(acc2, Y2, dis, b2p):
  def body(a_ref, y2_ref, dis_ref, b2_ref, lp_ref, o_ref):
    dis0 = dis_ref[:, 0:1]
    o = (a_ref[0] + a_ref[1] + y2_ref[...]) * dis0 + b2_ref[...]
    logits = o[:, :N_CLASSES]
    m = jnp.max(logits, axis=1, keepdims=True)
    lse = jnp.log(jnp.sum(jnp.exp(logits - m), axis=1, keepdims=True)) + m
    o_ref[...] = logits
    lp_ref[...] = logits - lse

  return pl.pallas_call(
      body,
      grid=(NBLK,),
      in_specs=[
          pl.BlockSpec((NCORES, RB, CPAD), lambda i: (0, i, 0)),
          pl.BlockSpec((RB, CPAD), lambda i: (i, 0)),
          pl.BlockSpec((RB, 16), lambda i: (i, 0)),
          pl.BlockSpec((1, CPAD), lambda i: (0, 0)),
      ],
      out_specs=[
          pl.BlockSpec((RB, N_CLASSES), lambda i: (i, 0)),
          pl.BlockSpec((RB, N_CLASSES), lambda i: (i, 0)),
      ],
      out_shape=[
          jax.ShapeDtypeStruct((N, N_CLASSES), jnp.float32),
          jax.ShapeDtypeStruct((N, N_CLASSES), jnp.float32),
      ],
  )(acc2, Y2, dis, b2p)


def kernel(x, edge_index, W1, b1, W2, b2):
  src = edge_index[0]
  dst = edge_index[1]
  pad = EPAD - E
  srcp = jnp.concatenate(
      [src, jnp.zeros((pad,), jnp.int32)]).reshape(EROWS, CHUNK)
  dstp = jnp.concatenate(
      [dst, jnp.full((pad,), N, jnp.int32)]).reshape(EROWS, CHUNK)

  ones16 = jnp.ones((CHUNK, 16), jnp.float32)
  z16 = jnp.zeros((RPT, 16), jnp.float32)
  z128 = jnp.zeros((RPT, EMBED), jnp.float32)
  z32 = jnp.zeros((RPT, CPAD), jnp.float32)

  degp = _deg(ones16, dstp, z16)
  dis, Y1 = _tc_a(x, W1, degp[:, :N, :])
  acc1 = _seg128(Y1, srcp, dstp, z128)[:, :N, :]

  W2p = jnp.pad(W2, ((0, 0), (0, CPAD - N_CLASSES)))
  b2p = jnp.pad(b2, (0, CPAD - N_CLASSES)).reshape(1, CPAD)
  Y2 = _tc_b(acc1, Y1, dis, b1.reshape(1, EMBED), W2p)
  acc2 = _seg32(Y2, srcp, dstp, z32)[:, :N, :]

  logp, out = _tc_c(acc2, Y2, dis, b2p)
  return (logp, out)


# speedup vs baseline: 16.4375x; 1.0311x over previous
import functools

import jax
import jax.numpy as jnp
from jax import lax
from jax.experimental import pallas as pl
from jax.experimental.pallas import tpu as pltpu
from jax.experimental.pallas import tpu_sc as plsc

N = 10000
E = 320000
D_FEAT = 128
EMBED = 128
N_CLASSES = 18
CPAD = 32

NCORES = 2
NSUB = 16
NW = NCORES * NSUB
CHUNK = 128
CPT = 81
EPT = CHUNK * CPT
EPAD = NW * EPT
EROWS = EPAD // CHUNK
NPAD = 10240
RPT = NPAD // NSUB

RB = 400
NBLK = N // RB

_mesh = plsc.VectorSubcoreMesh(core_axis_name="c", subcore_axis_name="s")


def _make_seg(width):
  @functools.partial(
      pl.kernel,
      mesh=_mesh,
      compiler_params=pltpu.CompilerParams(use_tc_tiling_on_sc=False),
      out_type=jax.ShapeDtypeStruct((NCORES, NPAD, width), jnp.bfloat16),
      scratch_types=[
          pltpu.VMEM((CPT, CHUNK), jnp.int32),
          pltpu.VMEM((CPT, CHUNK), jnp.int32),
          pltpu.VMEM((CHUNK, width), jnp.bfloat16),
          pltpu.VMEM((CHUNK, width), jnp.bfloat16),
          pltpu.VMEM((CHUNK, width), jnp.bfloat16),
          pltpu.SemaphoreType.DMA,
          pltpu.SemaphoreType.DMA,
          pltpu.SemaphoreType.DMA,
          pltpu.SemaphoreType.DMA,
          pltpu.SemaphoreType.DMA,
          pltpu.SemaphoreType.DMA,
          pltpu.VMEM_SHARED((NPAD, width), jnp.bfloat16),
      ],
  )
  def seg(y_hbm, src_hbm, dst_hbm, zero_hbm, out_hbm, idx_s, idx_d,
          d0, d1, d2, g0, g1, g2, s0, s1, s2, acc):
    c = lax.axis_index("c")
    s = lax.axis_index("s")
    w = c * NSUB + s
    pltpu.sync_copy(zero_hbm, acc.at[pl.ds(s * RPT, RPT)])
    pltpu.sync_copy(src_hbm.at[pl.ds(w * CPT, CPT)], idx_s)
    pltpu.sync_copy(dst_hbm.at[pl.ds(w * CPT, CPT)], idx_d)
    plsc.subcore_barrier()

    bufs = (d0, d1, d2)
    gsem = (g0, g1, g2)
    ssem = (s0, s1, s2)

    def gather(j, p):
      return pltpu.make_async_copy(y_hbm.at[idx_s.at[j]], bufs[p], gsem[p])

    def scatter_start(j, p):
      pltpu.async_copy(bufs[p], acc.at[idx_d.at[j]], ssem[p], add=True)

    def scatter_wait(j, p):
      pltpu.make_async_copy(bufs[p], acc.at[idx_d.at[j]], ssem[p]).wait()

    gather(0, 0).start()
    gather(1, 1).start()

    @pl.loop(0, CPT, step=3)
    def _(j):
      for p in range(3):
        jj = j + p
        gather(jj, p).wait()
        scatter_start(jj, p)
        q = (p + 2) % 3

        @pl.when(jj >= 1)
        def _():
          scatter_wait(jj - 1, q)

        @pl.when(jj + 2 < CPT)
        def _():
          gather(jj + 2, q).start()

    scatter_wait(CPT - 1, (CPT - 1) % 3)
    plsc.subcore_barrier()
    pltpu.sync_copy(acc.at[pl.ds(s * RPT, RPT)],
                    out_hbm.at[c, pl.ds(s * RPT, RPT)])

  return seg


_seg128 = _make_seg(EMBED)
_seg32 = _make_seg(CPAD)


@functools.partial(
    pl.kernel,
    mesh=_mesh,
    compiler_params=pltpu.CompilerParams(use_tc_tiling_on_sc=False),
    out_type=jax.ShapeDtypeStruct((NCORES, NPAD, 16), jnp.float32),
    scratch_types=[
        pltpu.VMEM((CPT, CHUNK), jnp.int32),
        pltpu.VMEM((CHUNK, 16), jnp.float32),
        pltpu.VMEM_SHARED((NPAD, 16), jnp.float32),
    ],
)
def _deg(ones_hbm, dst_hbm, zero_hbm, out_hbm, idx_d, ones_v, acc):
  c = lax.axis_index("c")
  s = lax.axis_index("s")
  w = c * NSUB + s
  pltpu.sync_copy(zero_hbm, acc.at[pl.ds(s * RPT, RPT)])
  pltpu.sync_copy(ones_hbm, ones_v)
  pltpu.sync_copy(dst_hbm.at[pl.ds(w * CPT, CPT)], idx_d)
  plsc.subcore_barrier()

  @pl.loop(0, CPT)
  def _(j):
    pltpu.sync_copy(ones_v, acc.at[idx_d.at[j]], add=True)

  plsc.subcore_barrier()
  pltpu.sync_copy(acc.at[pl.ds(s * RPT, RPT)],
                  out_hbm.at[c, pl.ds(s * RPT, RPT)])


def _tc_a(x, W1, degp):
  def body(x_ref, w_ref, d_ref, dis_ref, y_ref):
    deg = d_ref[0] + d_ref[1] + 1.0
    dis = lax.rsqrt(deg)
    dis_ref[...] = dis
    xw = jnp.dot(x_ref[...], w_ref[...], preferred_element_type=jnp.float32)
    y_ref[...] = (xw * dis[:, 0:1]).astype(jnp.bfloat16)

  return pl.pallas_call(
      body,
      grid=(NBLK,),
      in_specs=[
          pl.BlockSpec((RB, D_FEAT), lambda i: (i, 0)),
          pl.BlockSpec((D_FEAT, EMBED), lambda i: (0, 0)),
          pl.BlockSpec((NCORES, RB, 16), lambda i: (0, i, 0)),
      ],
      out_specs=[
          pl.BlockSpec((RB, 16), lambda i: (i, 0)),
          pl.BlockSpec((RB, EMBED), lambda i: (i, 0)),
      ],
      out_shape=[
          jax.ShapeDtypeStruct((N, 16), jnp.float32),
          jax.ShapeDtypeStruct((N, EMBED), jnp.bfloat16),
      ],
  )(x, W1, degp)


def _tc_b(acc1, Y1, dis, b1, W2p):
  def body(a_ref, y1_ref, dis_ref, b1_ref, w2_ref, y2_ref):
    dis0 = dis_ref[:, 0:1]
    agg = (a_ref[0] + a_ref[1] + y1_ref[...]).astype(jnp.float32)
    h = jnp.maximum(agg * dis0 + b1_ref[...], 0.0)
    y2 = jnp.dot(h, w2_ref[...], preferred_element_type=jnp.float32) * dis0
    y2_ref[...] = y2.astype(jnp.bfloat16)

  return pl.pallas_call(
      body,
      grid=(NBLK,),
      in_specs=[
          pl.BlockSpec((NCORES, RB, EMBED), lambda i: (0, i, 0)),
          pl.BlockSpec((RB, EMBED), lambda i: (i, 0)),
          pl.BlockSpec((RB, 16), lambda i: (i, 0)),
          pl.BlockSpec((1, EMBED), lambda i: (0, 0)),
          pl.BlockSpec((EMBED, CPAD), lambda i: (0, 0)),
      ],
      out_specs=pl.BlockSpec((RB, CPAD), lambda i: (i, 0)),
      out_shape=jax.ShapeDtypeStruct((N, CPAD), jnp.bfloat16),
  )(acc1, Y1, dis, b1, W2p)


def _tc_c(acc2, Y2, dis, b2p):
  def body(a_ref, y2_ref, dis_ref, b2_ref, lp_ref, o_ref):
    dis0 = dis_ref[:, 0:1]
    agg = (a_ref[0] + a_ref[1] + y2_ref[...]).astype(jnp.float32)
    o = agg * dis0 + b2_ref[...]
    logits = o[:, :N_CLASSES]
    m = jnp.max(logits, axis=1, keepdims=True)
    lse = jnp.log(jnp.sum(jnp.exp(logits - m), axis=1, keepdims=True)) + m
    o_ref[...] = logits
    lp_ref[...] = logits - lse

  return pl.pallas_call(
      body,
      grid=(NBLK,),
      in_specs=[
          pl.BlockSpec((NCORES, RB, CPAD), lambda i: (0, i, 0)),
          pl.BlockSpec((RB, CPAD), lambda i: (i, 0)),
          pl.BlockSpec((RB, 16), lambda i: (i, 0)),
          pl.BlockSpec((1, CPAD), lambda i: (0, 0)),
      ],
      out_specs=[
          pl.BlockSpec((RB, N_CLASSES), lambda i: (i, 0)),
          pl.BlockSpec((RB, N_CLASSES), lambda i: (i, 0)),
      ],
      out_shape=[
          jax.ShapeDtypeStruct((N, N_CLASSES), jnp.float32),
          jax.ShapeDtypeStruct((N, N_CLASSES), jnp.float32),
      ],
  )(acc2, Y2, dis, b2p)


def kernel(x, edge_index, W1, b1, W2, b2):
  src = edge_index[0]
  dst = edge_index[1]
  pad = EPAD - E
  srcp = jnp.concatenate(
      [src, jnp.zeros((pad,), jnp.int32)]).reshape(EROWS, CHUNK)
  dstp = jnp.concatenate(
      [dst, jnp.full((pad,), N, jnp.int32)]).reshape(EROWS, CHUNK)

  ones16 = jnp.ones((CHUNK, 16), jnp.float32)
  z16 = jnp.zeros((RPT, 16), jnp.float32)
  z128 = jnp.zeros((RPT, EMBED), jnp.bfloat16)
  z32 = jnp.zeros((RPT, CPAD), jnp.bfloat16)

  degp = _deg(ones16, dstp, z16)
  dis, Y1 = _tc_a(x, W1, degp[:, :N, :])
  acc1 = _seg128(Y1, srcp, dstp, z128)[:, :N, :]

  W2p = jnp.pad(W2, ((0, 0), (0, CPAD - N_CLASSES)))
  b2p = jnp.pad(b2, (0, CPAD - N_CLASSES)).reshape(1, CPAD)
  Y2 = _tc_b(acc1, Y1, dis, b1.reshape(1, EMBED), W2p)
  acc2 = _seg32(Y2, srcp, dstp, z32)[:, :N, :]

  logp, out = _tc_c(acc2, Y2, dis, b2p)
  return (logp, out)
